# baseline (speedup 1.0000x reference)
import jax
import jax.numpy as jnp
from jax import lax
from jax.experimental import pallas as pl

_EMB = 64
_K = 512
_SPARSITY = 5
_BLK = 2048

_DN = (((1,), (0,)), ((), ()))


def _mm(a, b, prec):
    return lax.dot_general(a, b, _DN, precision=prec,
                           preferred_element_type=jnp.float32)


def _omp_half(St, dn, corr0, splits):
    B = St.shape[1]
    g1, g2, g3 = splits
    iota = lax.broadcasted_iota(jnp.int32, (_K, B), 0)
    pen = jnp.zeros((_K, B), jnp.float32)
    corr = corr0
    idxs = []
    rows = []
    vals = []
    L = {(0, 0): jnp.ones((1, B), jnp.float32)}
    c = []
    for t in range(_SPARSITY):
        a = jnp.abs(corr) - pen
        m = jnp.max(a, axis=0, keepdims=True)
        idx = jnp.min(jnp.where(a >= m, iota, _K), axis=0, keepdims=True)
        hot = (iota == idx).astype(jnp.float32)
        if t < _SPARSITY - 1:
            pen = pen + hot * 1e30
        val = jnp.sum(hot * corr0, axis=0, keepdims=True)
        idxs.append(idx)
        vals.append(val)
        if t > 0:
            g = [jnp.sum(hot * rows[i], axis=0, keepdims=True)
                 for i in range(t)]
            w = []
            for i in range(t):
                acc = g[i]
                for j in range(i):
                    acc = acc - L[(i, j)] * w[j]
                w.append(acc / L[(i, i)])
            for j in range(t):
                L[(t, j)] = w[j]
            L[(t, t)] = jnp.sqrt(1.0 - sum(wj * wj for wj in w))
        n = t + 1
        y = []
        for i in range(n):
            acc = vals[i]
            for j in range(i):
                acc = acc - L[(i, j)] * y[j]
            y.append(acc / L[(i, i)])
        c = [None] * n
        for i in reversed(range(n)):
            acc = y[i]
            for j in range(i + 1, n):
                acc = acc - L[(j, i)] * c[j]
            c[i] = acc / L[(i, i)]
        if t < _SPARSITY - 1:
            hotb = hot.astype(jnp.bfloat16)
            row = (_mm(g1, hotb, None) + _mm(g2, hotb, None)
                   + _mm(g3, hotb, None))
            rows.append(row)
            beta = c[0] * rows[0]
            for i in range(1, n):
                beta = beta + c[i] * rows[i]
            corr = corr0 - beta
    coeff = c[0] * (iota == idxs[0]).astype(jnp.float32)
    for i in range(1, _SPARSITY):
        coeff = coeff + c[i] * (iota == idxs[i]).astype(jnp.float32)
    zdl = _mm(dn, coeff, None)
    return coeff, zdl


def _omp_body(st_ref, dn_ref, dnt_ref, coeff_ref, zdl_ref, loss_ref):
    B = st_ref.shape[1]
    dn = dn_ref[...]
    dnt = dnt_ref[...]

    G = _mm(dnt, dn, None)
    g1f = G.astype(jnp.bfloat16).astype(jnp.float32)
    g2f = (G - g1f).astype(jnp.bfloat16).astype(jnp.float32)
    g3 = (G - g1f - g2f).astype(jnp.bfloat16)
    splits = (g1f.astype(jnp.bfloat16), g2f.astype(jnp.bfloat16), g3)

    h = B // 2
    part = None
    for s in range(2):
        St = st_ref[:, pl.ds(s * h, h)]
        corr0 = _mm(dnt, St, None)
        coeff, zdl = _omp_half(St, dn, corr0, splits)
        coeff_ref[:, pl.ds(s * h, h)] = coeff
        zdl_ref[:, pl.ds(s * h, h)] = zdl
        diff = zdl - St
        p = jnp.sum(jnp.sum(diff * diff, axis=1, keepdims=True),
                    axis=0, keepdims=True)
        part = p if part is None else part + p

    @pl.when(pl.program_id(0) == 0)
    def _init():
        loss_ref[...] = jnp.zeros_like(loss_ref)

    loss_ref[...] += part


@jax.jit
def _run(z_e, dictionary):
    bsz, ch, hh, ww = z_e.shape
    n = bsz * hh * ww
    ze_flat = jnp.transpose(z_e, (0, 2, 3, 1)).reshape(ch, n)
    dn = dictionary / jnp.linalg.norm(dictionary, axis=0)
    dnt = dn.T
    blk = min(_BLK, n)
    nb = n // blk
    coeff, zdl, loss_sum = pl.pallas_call(
        _omp_body,
        grid=(nb,),
        in_specs=[
            pl.BlockSpec((ch, blk), lambda i: (0, i)),
            pl.BlockSpec((ch, _K), lambda i: (0, 0)),
            pl.BlockSpec((_K, ch), lambda i: (0, 0)),
        ],
        out_specs=[
            pl.BlockSpec((_K, blk), lambda i: (0, i)),
            pl.BlockSpec((ch, blk), lambda i: (0, i)),
            pl.BlockSpec((1, 1), lambda i: (0, 0)),
        ],
        out_shape=[
            jax.ShapeDtypeStruct((_K, n), jnp.float32),
            jax.ShapeDtypeStruct((ch, n), jnp.float32),
            jax.ShapeDtypeStruct((1, 1), jnp.float32),
        ],
    )(ze_flat, dn, dnt)
    out = jnp.transpose(zdl.reshape(bsz, hh, ww, ch), (0, 3, 1, 2))
    loss = 1.25 * loss_sum[0, 0] / (n * ch)
    return out, loss, coeff


def kernel(z_e, dictionary):
    return _run(z_e, dictionary)


# speedup vs baseline: 1.0182x; 1.0182x over previous
import jax
import jax.numpy as jnp
from jax import lax
from jax.experimental import pallas as pl

_EMB = 64
_K = 512
_SPARSITY = 5
_BLK = 2048

_DN = (((1,), (0,)), ((), ()))


def _mm(a, b, prec):
    return lax.dot_general(a, b, _DN, precision=prec,
                           preferred_element_type=jnp.float32)


def _omp_body(st_ref, dn_ref, dnt_ref, coeff_ref, zdl_ref, loss_ref):
    B = st_ref.shape[1]
    St = st_ref[...]
    dn = dn_ref[...]
    dnt = dnt_ref[...]

    G = _mm(dnt, dn, None)
    corr0 = _mm(dnt, St, None)
    g1f = G.astype(jnp.bfloat16).astype(jnp.float32)
    g2f = (G - g1f).astype(jnp.bfloat16).astype(jnp.float32)
    g3 = (G - g1f - g2f).astype(jnp.bfloat16)
    g1 = g1f.astype(jnp.bfloat16)
    g2 = g2f.astype(jnp.bfloat16)
    iota = lax.broadcasted_iota(jnp.int32, (_K, B), 0)
    pen = jnp.zeros((_K, B), jnp.float32)
    corr = corr0
    idxs = []
    rows = []
    vals = []
    L = {(0, 0): jnp.ones((1, B), jnp.float32)}
    c = []
    for t in range(_SPARSITY):
        a = jnp.abs(corr) - pen
        m = jnp.max(a, axis=0, keepdims=True)
        idx = jnp.min(jnp.where(a >= m, iota, _K), axis=0, keepdims=True)
        hot = (iota == idx).astype(jnp.float32)
        if t < _SPARSITY - 1:
            pen = pen + hot * 1e30
        val = jnp.sum(hot * corr0, axis=0, keepdims=True)
        idxs.append(idx)
        vals.append(val)
        if t > 0:
            g = [jnp.sum(hot * rows[i], axis=0, keepdims=True)
                 for i in range(t)]
            w = []
            for i in range(t):
                acc = g[i]
                for j in range(i):
                    acc = acc - L[(i, j)] * w[j]
                w.append(acc / L[(i, i)])
            for j in range(t):
                L[(t, j)] = w[j]
            L[(t, t)] = jnp.sqrt(1.0 - sum(wj * wj for wj in w))
        n = t + 1
        y = []
        for i in range(n):
            acc = vals[i]
            for j in range(i):
                acc = acc - L[(i, j)] * y[j]
            y.append(acc / L[(i, i)])
        c = [None] * n
        for i in reversed(range(n)):
            acc = y[i]
            for j in range(i + 1, n):
                acc = acc - L[(j, i)] * c[j]
            c[i] = acc / L[(i, i)]
        if t < _SPARSITY - 1:
            hotb = hot.astype(jnp.bfloat16)
            row = (_mm(g1, hotb, None) + _mm(g2, hotb, None)
                   + _mm(g3, hotb, None))
            rows.append(row)
            beta = c[0] * rows[0]
            for i in range(1, n):
                beta = beta + c[i] * rows[i]
            corr = corr0 - beta
    coeff = c[0] * (iota == idxs[0]).astype(jnp.float32)
    for i in range(1, _SPARSITY):
        coeff = coeff + c[i] * (iota == idxs[i]).astype(jnp.float32)
    coeff_ref[...] = coeff
    zdl = _mm(dn, coeff, None)
    zdl_ref[...] = zdl
    diff = zdl - St
    part = jnp.sum(jnp.sum(diff * diff, axis=1, keepdims=True),
                   axis=0, keepdims=True)

    @pl.when(pl.program_id(0) == 0)
    def _init():
        loss_ref[...] = jnp.zeros_like(loss_ref)

    loss_ref[...] += part


@jax.jit
def _run(z_e, dictionary):
    bsz, ch, hh, ww = z_e.shape
    n = bsz * hh * ww
    ze_flat = jnp.transpose(z_e, (0, 2, 3, 1)).reshape(ch, n)
    dn = dictionary / jnp.linalg.norm(dictionary, axis=0)
    dnt = dn.T
    blk = min(_BLK, n)
    nb = n // blk
    coeff, zdl, loss_sum = pl.pallas_call(
        _omp_body,
        grid=(nb,),
        in_specs=[
            pl.BlockSpec((ch, blk), lambda i: (0, i)),
            pl.BlockSpec((ch, _K), lambda i: (0, 0)),
            pl.BlockSpec((_K, ch), lambda i: (0, 0)),
        ],
        out_specs=[
            pl.BlockSpec((_K, blk), lambda i: (0, i)),
            pl.BlockSpec((ch, blk), lambda i: (0, i)),
            pl.BlockSpec((1, 1), lambda i: (0, 0)),
        ],
        out_shape=[
            jax.ShapeDtypeStruct((_K, n), jnp.float32),
            jax.ShapeDtypeStruct((ch, n), jnp.float32),
            jax.ShapeDtypeStruct((1, 1), jnp.float32),
        ],
    )(ze_flat, dn, dnt)
    out = jnp.transpose(zdl.reshape(bsz, hh, ww, ch), (0, 3, 1, 2))
    loss = 1.25 * loss_sum[0, 0] / (n * ch)
    return out, loss, coeff


def kernel(z_e, dictionary):
    return _run(z_e, dictionary)


# speedup vs baseline: 1.1412x; 1.1208x over previous
import jax
import jax.numpy as jnp
from jax import lax
from jax.experimental import pallas as pl

_EMB = 64
_K = 512
_SPARSITY = 5
_BLK = 2048

_DN = (((1,), (0,)), ((), ()))


def _mm(a, b, prec):
    return lax.dot_general(a, b, _DN, precision=prec,
                           preferred_element_type=jnp.float32)


def _omp_body(st_ref, dn_ref, dnt_ref, coeff_ref, zdl_ref, loss_ref):
    B = st_ref.shape[1]
    St = st_ref[...]
    dn = dn_ref[...]
    dnt = dnt_ref[...]

    G = _mm(dnt, dn, None)
    corr0 = _mm(dnt, St, None)
    g1f = G.astype(jnp.bfloat16).astype(jnp.float32)
    g2f = (G - g1f).astype(jnp.bfloat16).astype(jnp.float32)
    g3 = (G - g1f - g2f).astype(jnp.bfloat16)
    g1 = g1f.astype(jnp.bfloat16)
    g2 = g2f.astype(jnp.bfloat16)
    iota = lax.broadcasted_iota(jnp.int32, (_K, B), 0)
    pen = jnp.zeros((_K, B), jnp.float32)
    corr = corr0
    idxs = []
    rows = []
    vals = []
    L = {(0, 0): jnp.ones((1, B), jnp.float32)}
    c = []
    for t in range(_SPARSITY):
        a = jnp.abs(corr) - pen
        idx = jnp.argmax(a, axis=0)[None, :].astype(jnp.int32)
        hot = (iota == idx).astype(jnp.float32)
        if t < _SPARSITY - 1:
            pen = pen + hot * 1e30
        val = jnp.sum(hot * corr0, axis=0, keepdims=True)
        idxs.append(idx)
        vals.append(val)
        if t > 0:
            g = [jnp.sum(hot * rows[i], axis=0, keepdims=True)
                 for i in range(t)]
            w = []
            for i in range(t):
                acc = g[i]
                for j in range(i):
                    acc = acc - L[(i, j)] * w[j]
                w.append(acc / L[(i, i)])
            for j in range(t):
                L[(t, j)] = w[j]
            L[(t, t)] = jnp.sqrt(1.0 - sum(wj * wj for wj in w))
        n = t + 1
        y = []
        for i in range(n):
            acc = vals[i]
            for j in range(i):
                acc = acc - L[(i, j)] * y[j]
            y.append(acc / L[(i, i)])
        c = [None] * n
        for i in reversed(range(n)):
            acc = y[i]
            for j in range(i + 1, n):
                acc = acc - L[(j, i)] * c[j]
            c[i] = acc / L[(i, i)]
        if t < _SPARSITY - 1:
            hotb = hot.astype(jnp.bfloat16)
            row = (_mm(g1, hotb, None) + _mm(g2, hotb, None)
                   + _mm(g3, hotb, None))
            rows.append(row)
            beta = c[0] * rows[0]
            for i in range(1, n):
                beta = beta + c[i] * rows[i]
            corr = corr0 - beta
    coeff = c[0] * (iota == idxs[0]).astype(jnp.float32)
    for i in range(1, _SPARSITY):
        coeff = coeff + c[i] * (iota == idxs[i]).astype(jnp.float32)
    coeff_ref[...] = coeff
    zdl = _mm(dn, coeff, None)
    zdl_ref[...] = zdl
    diff = zdl - St
    part = jnp.sum(jnp.sum(diff * diff, axis=1, keepdims=True),
                   axis=0, keepdims=True)

    @pl.when(pl.program_id(0) == 0)
    def _init():
        loss_ref[...] = jnp.zeros_like(loss_ref)

    loss_ref[...] += part


@jax.jit
def _run(z_e, dictionary):
    bsz, ch, hh, ww = z_e.shape
    n = bsz * hh * ww
    ze_flat = jnp.transpose(z_e, (0, 2, 3, 1)).reshape(ch, n)
    dn = dictionary / jnp.linalg.norm(dictionary, axis=0)
    dnt = dn.T
    blk = min(_BLK, n)
    nb = n // blk
    coeff, zdl, loss_sum = pl.pallas_call(
        _omp_body,
        grid=(nb,),
        in_specs=[
            pl.BlockSpec((ch, blk), lambda i: (0, i)),
            pl.BlockSpec((ch, _K), lambda i: (0, 0)),
            pl.BlockSpec((_K, ch), lambda i: (0, 0)),
        ],
        out_specs=[
            pl.BlockSpec((_K, blk), lambda i: (0, i)),
            pl.BlockSpec((ch, blk), lambda i: (0, i)),
            pl.BlockSpec((1, 1), lambda i: (0, 0)),
        ],
        out_shape=[
            jax.ShapeDtypeStruct((_K, n), jnp.float32),
            jax.ShapeDtypeStruct((ch, n), jnp.float32),
            jax.ShapeDtypeStruct((1, 1), jnp.float32),
        ],
    )(ze_flat, dn, dnt)
    out = jnp.transpose(zdl.reshape(bsz, hh, ww, ch), (0, 3, 1, 2))
    loss = 1.25 * loss_sum[0, 0] / (n * ch)
    return out, loss, coeff


def kernel(z_e, dictionary):
    return _run(z_e, dictionary)
